# Initial kernel scaffold; baseline (speedup 1.0000x reference)
import jax
import jax.numpy as jnp
from jax.experimental import pallas as pl


def kernel(edge_attr_0, edge_attr_1, edge_attr_2, edge_index_0, edge_index_1, num_nodes):
    raise NotImplementedError("write your pallas kernel here")


# speedup vs baseline: 4.0917x; 4.0917x over previous
import functools

import jax
import jax.numpy as jnp
from jax import lax
from jax.experimental import pallas as pl
from jax.experimental.pallas import tpu as pltpu
from jax.experimental.pallas import tpu_sc as plsc

N_NODES = 10000
N_EDGES = 320000
D = 128

NC = 2
NS = 16
NW = NC * NS

EPW = N_EDGES // NW
CH = 80
NCHUNK = EPW // CH
TILE_ROWS = 640
LAST_TILE_ROWS = N_NODES - (NS - 1) * TILE_ROWS


def _sc_scatter(ea1, ea2, idx0, idx1):
    mesh = plsc.VectorSubcoreMesh(core_axis_name="c", subcore_axis_name="s")

    @functools.partial(
        pl.kernel,
        mesh=mesh,
        out_type=jax.ShapeDtypeStruct((2 * N_NODES, D), jnp.float32),
        scratch_types=[
            pltpu.VMEM_SHARED((N_NODES, D), jnp.float32),
            pltpu.VMEM((NCHUNK, CH), jnp.int32),
            pltpu.VMEM((CH, D), jnp.float32),
        ],
    )
    def body(ea1_hbm, ea2_hbm, idx0_hbm, idx1_hbm, out_hbm, acc, idx_v, rows_v):
        c = lax.axis_index("c")
        s = lax.axis_index("s")
        w = s * NC + c

        def zrow(i, carry):
            def zcol(k, carry2):
                rows_v[i, pl.ds(k * 16, 16)] = jnp.zeros((16,), jnp.float32)
                return carry2
            return lax.fori_loop(0, D // 16, zcol, carry)
        lax.fori_loop(0, CH, zrow, 0)

        @pl.when(s < NS - 1)
        def _():
            for t in range(TILE_ROWS // CH):
                pltpu.sync_copy(rows_v, acc.at[pl.ds(s * TILE_ROWS + t * CH, CH)])

        @pl.when(s == NS - 1)
        def _():
            for t in range(LAST_TILE_ROWS // CH):
                pltpu.sync_copy(
                    rows_v, acc.at[pl.ds((NS - 1) * TILE_ROWS + t * CH, CH)])

        plsc.subcore_barrier()

        def phase(ea_hbm, idx_hbm):
            pltpu.sync_copy(idx_hbm.at[w], idx_v)

            def chunk(j, carry):
                pltpu.sync_copy(ea_hbm.at[pl.ds(w * EPW + j * CH, CH)], rows_v)
                pltpu.sync_copy(rows_v, acc.at[idx_v.at[j]], add=True)
                return carry
            lax.fori_loop(0, NCHUNK, chunk, 0)

        phase(ea1_hbm, idx0_hbm)
        phase(ea2_hbm, idx1_hbm)
        plsc.subcore_barrier()

        @pl.when(s < NS - 1)
        def _():
            pltpu.sync_copy(
                acc.at[pl.ds(s * TILE_ROWS, TILE_ROWS)],
                out_hbm.at[pl.ds(c * N_NODES + s * TILE_ROWS, TILE_ROWS)],
            )

        @pl.when(s == NS - 1)
        def _():
            pltpu.sync_copy(
                acc.at[pl.ds((NS - 1) * TILE_ROWS, LAST_TILE_ROWS)],
                out_hbm.at[pl.ds(c * N_NODES + (NS - 1) * TILE_ROWS, LAST_TILE_ROWS)],
            )

    return body(ea1, ea2, idx0, idx1)


_MERGE_BL = 1000


def _tc_merge(partials, edge_attr_0, mfac):
    def body(m_ref, p0_ref, p1_ref, ea0_ref, o_ref):
        o_ref[...] = (p0_ref[...] + p1_ref[...]) * m_ref[0] + ea0_ref[...]

    nblk = N_NODES // _MERGE_BL
    return pl.pallas_call(
        body,
        grid=(nblk,),
        in_specs=[
            pl.BlockSpec(memory_space=pltpu.SMEM),
            pl.BlockSpec((_MERGE_BL, D), lambda i: (i, 0)),
            pl.BlockSpec((_MERGE_BL, D), lambda i: (i + nblk, 0)),
            pl.BlockSpec((_MERGE_BL, D), lambda i: (i, 0)),
        ],
        out_specs=pl.BlockSpec((_MERGE_BL, D), lambda i: (i, 0)),
        out_shape=jax.ShapeDtypeStruct((N_NODES, D), jnp.float32),
    )(mfac, partials, partials, edge_attr_0)


def kernel(edge_attr_0, edge_attr_1, edge_attr_2, edge_index_0, edge_index_1, num_nodes):
    idx0 = edge_index_0[0].reshape(NW, NCHUNK, CH)
    idx1 = edge_index_1[0].reshape(NW, NCHUNK, CH)
    partials = _sc_scatter(edge_attr_1, edge_attr_2, idx0, idx1)
    mfac = (jnp.asarray(num_nodes, jnp.int32) // N_NODES).astype(jnp.float32).reshape(1)
    return _tc_merge(partials, edge_attr_0, mfac)


# speedup vs baseline: 7.8362x; 1.9151x over previous
import functools

import jax
import jax.numpy as jnp
from jax import lax
from jax.experimental import pallas as pl
from jax.experimental.pallas import tpu as pltpu
from jax.experimental.pallas import tpu_sc as plsc

N_NODES = 10000
N_EDGES = 320000
D = 128

NC = 2
NS = 16
NW = NC * NS

EPW = N_EDGES // NW
CH = 80
NCHUNK = EPW // CH
NBUF = 3
PIPE = (NCHUNK // NBUF) * NBUF
TILE_ROWS = 640
LAST_TILE_ROWS = N_NODES - (NS - 1) * TILE_ROWS


def _sc_scatter(ea1, ea2, idx0, idx1):
    mesh = plsc.VectorSubcoreMesh(core_axis_name="c", subcore_axis_name="s")

    @functools.partial(
        pl.kernel,
        mesh=mesh,
        out_type=jax.ShapeDtypeStruct((2 * N_NODES, D), jnp.float32),
        scratch_types=[
            pltpu.VMEM_SHARED((N_NODES, D), jnp.float32),
            pltpu.VMEM((NCHUNK, CH), jnp.int32),
            pltpu.VMEM((NBUF, CH, D), jnp.float32),
        ] + [pltpu.SemaphoreType.DMA] * NBUF,
    )
    def body(ea1_hbm, ea2_hbm, idx0_hbm, idx1_hbm, out_hbm, acc, idx_v, rows_v,
             *sems):
        c = lax.axis_index("c")
        s = lax.axis_index("s")
        w = s * NC + c

        def zrow(i, carry):
            def zcol(k, carry2):
                rows_v[0, i, pl.ds(k * 16, 16)] = jnp.zeros((16,), jnp.float32)
                return carry2
            return lax.fori_loop(0, D // 16, zcol, carry)
        lax.fori_loop(0, CH, zrow, 0)

        @pl.when(s < NS - 1)
        def _():
            for t in range(TILE_ROWS // CH):
                pltpu.sync_copy(
                    rows_v.at[0], acc.at[pl.ds(s * TILE_ROWS + t * CH, CH)])

        @pl.when(s == NS - 1)
        def _():
            for t in range(LAST_TILE_ROWS // CH):
                pltpu.sync_copy(
                    rows_v.at[0], acc.at[pl.ds((NS - 1) * TILE_ROWS + t * CH, CH)])

        plsc.subcore_barrier()

        def load(ea_hbm, j, b):
            return pltpu.make_async_copy(
                ea_hbm.at[pl.ds(w * EPW + j * CH, CH)], rows_v.at[b], sems[b])

        def phase(ea_hbm, idx_hbm):
            pltpu.sync_copy(idx_hbm.at[w], idx_v)
            for b in range(NBUF):
                load(ea_hbm, b, b).start()

            def outer(g, carry):
                for b in range(NBUF):
                    j = g * NBUF + b
                    load(ea_hbm, j, b).wait()
                    pltpu.sync_copy(rows_v.at[b], acc.at[idx_v.at[j]], add=True)
                    jn = j + NBUF

                    @pl.when(jn < PIPE)
                    def _():
                        load(ea_hbm, jn, b).start()
                return carry
            lax.fori_loop(0, PIPE // NBUF, outer, 0)

            for j in range(PIPE, NCHUNK):
                pltpu.sync_copy(ea_hbm.at[pl.ds(w * EPW + j * CH, CH)],
                                rows_v.at[0])
                pltpu.sync_copy(rows_v.at[0], acc.at[idx_v.at[j]], add=True)

        phase(ea1_hbm, idx0_hbm)
        phase(ea2_hbm, idx1_hbm)
        plsc.subcore_barrier()

        @pl.when(s < NS - 1)
        def _():
            pltpu.sync_copy(
                acc.at[pl.ds(s * TILE_ROWS, TILE_ROWS)],
                out_hbm.at[pl.ds(c * N_NODES + s * TILE_ROWS, TILE_ROWS)],
            )

        @pl.when(s == NS - 1)
        def _():
            pltpu.sync_copy(
                acc.at[pl.ds((NS - 1) * TILE_ROWS, LAST_TILE_ROWS)],
                out_hbm.at[pl.ds(c * N_NODES + (NS - 1) * TILE_ROWS, LAST_TILE_ROWS)],
            )

    return body(ea1, ea2, idx0, idx1)


_MERGE_BL = 1000


def _tc_merge(partials, edge_attr_0, mfac):
    def body(m_ref, p0_ref, p1_ref, ea0_ref, o_ref):
        o_ref[...] = (p0_ref[...] + p1_ref[...]) * m_ref[0] + ea0_ref[...]

    nblk = N_NODES // _MERGE_BL
    return pl.pallas_call(
        body,
        grid=(nblk,),
        in_specs=[
            pl.BlockSpec(memory_space=pltpu.SMEM),
            pl.BlockSpec((_MERGE_BL, D), lambda i: (i, 0)),
            pl.BlockSpec((_MERGE_BL, D), lambda i: (i + nblk, 0)),
            pl.BlockSpec((_MERGE_BL, D), lambda i: (i, 0)),
        ],
        out_specs=pl.BlockSpec((_MERGE_BL, D), lambda i: (i, 0)),
        out_shape=jax.ShapeDtypeStruct((N_NODES, D), jnp.float32),
    )(mfac, partials, partials, edge_attr_0)


def kernel(edge_attr_0, edge_attr_1, edge_attr_2, edge_index_0, edge_index_1, num_nodes):
    idx0 = edge_index_0[0].reshape(NW, NCHUNK, CH)
    idx1 = edge_index_1[0].reshape(NW, NCHUNK, CH)
    partials = _sc_scatter(edge_attr_1, edge_attr_2, idx0, idx1)
    mfac = (jnp.asarray(num_nodes, jnp.int32) // N_NODES).astype(jnp.float32).reshape(1)
    return _tc_merge(partials, edge_attr_0, mfac)


# speedup vs baseline: 7.9810x; 1.0185x over previous
import functools

import jax
import jax.numpy as jnp
from jax import lax
from jax.experimental import pallas as pl
from jax.experimental.pallas import tpu as pltpu
from jax.experimental.pallas import tpu_sc as plsc

N_NODES = 10000
N_EDGES = 320000
D = 128

NC = 2
NS = 16
NW = NC * NS

EPW = N_EDGES // NW
CH = 80
NCHUNK = EPW // CH
NBUF = 3
PIPE = (NCHUNK // NBUF) * NBUF
TILE_ROWS = 640
LAST_TILE_ROWS = N_NODES - (NS - 1) * TILE_ROWS
ZROWS = 16


def _sc_scatter(ea1, ea2, idx0, idx1):
    mesh = plsc.VectorSubcoreMesh(core_axis_name="c", subcore_axis_name="s")

    @functools.partial(
        pl.kernel,
        mesh=mesh,
        out_type=jax.ShapeDtypeStruct((2 * N_NODES, D), jnp.float32),
        scratch_types=[
            pltpu.VMEM_SHARED((N_NODES, D), jnp.float32),
            pltpu.VMEM((NCHUNK, CH), jnp.int32),
            pltpu.VMEM((NBUF, CH, D), jnp.float32),
            pltpu.VMEM((ZROWS, D), jnp.float32),
            pltpu.SemaphoreType.DMA,
        ] + [pltpu.SemaphoreType.DMA] * NBUF,
    )
    def body(ea1_hbm, ea2_hbm, idx0_hbm, idx1_hbm, out_hbm, acc, idx_v, rows_v,
             zbuf, sem_idx, *sems):
        c = lax.axis_index("c")
        s = lax.axis_index("s")
        w = s * NC + c

        def load(ea_hbm, j, b):
            return pltpu.make_async_copy(
                ea_hbm.at[pl.ds(w * EPW + j * CH, CH)], rows_v.at[b], sems[b])

        idx_cp0 = pltpu.make_async_copy(idx0_hbm.at[w], idx_v, sem_idx)
        idx_cp0.start()
        for b in range(NBUF):
            load(ea1_hbm, b, b).start()

        def zrow(i, carry):
            def zcol(k, carry2):
                zbuf[i, pl.ds(k * 16, 16)] = jnp.zeros((16,), jnp.float32)
                return carry2
            return lax.fori_loop(0, D // 16, zcol, carry)
        lax.fori_loop(0, ZROWS, zrow, 0)

        @pl.when(s < NS - 1)
        def _():
            for t in range(TILE_ROWS // ZROWS):
                pltpu.sync_copy(
                    zbuf, acc.at[pl.ds(s * TILE_ROWS + t * ZROWS, ZROWS)])

        @pl.when(s == NS - 1)
        def _():
            for t in range(LAST_TILE_ROWS // ZROWS):
                pltpu.sync_copy(
                    zbuf, acc.at[pl.ds((NS - 1) * TILE_ROWS + t * ZROWS, ZROWS)])

        plsc.subcore_barrier()
        idx_cp0.wait()

        def pipeline(ea_hbm):
            def outer(g, carry):
                for b in range(NBUF):
                    j = g * NBUF + b
                    load(ea_hbm, j, b).wait()
                    pltpu.sync_copy(rows_v.at[b], acc.at[idx_v.at[j]], add=True)
                    jn = j + NBUF

                    @pl.when(jn < PIPE)
                    def _():
                        load(ea_hbm, jn, b).start()
                return carry
            lax.fori_loop(0, PIPE // NBUF, outer, 0)

            for j in range(PIPE, NCHUNK):
                pltpu.sync_copy(ea_hbm.at[pl.ds(w * EPW + j * CH, CH)],
                                rows_v.at[0])
                pltpu.sync_copy(rows_v.at[0], acc.at[idx_v.at[j]], add=True)

        pipeline(ea1_hbm)
        pltpu.sync_copy(idx1_hbm.at[w], idx_v)
        for b in range(NBUF):
            load(ea2_hbm, b, b).start()
        pipeline(ea2_hbm)
        plsc.subcore_barrier()

        @pl.when(s < NS - 1)
        def _():
            pltpu.sync_copy(
                acc.at[pl.ds(s * TILE_ROWS, TILE_ROWS)],
                out_hbm.at[pl.ds(c * N_NODES + s * TILE_ROWS, TILE_ROWS)],
            )

        @pl.when(s == NS - 1)
        def _():
            pltpu.sync_copy(
                acc.at[pl.ds((NS - 1) * TILE_ROWS, LAST_TILE_ROWS)],
                out_hbm.at[pl.ds(c * N_NODES + (NS - 1) * TILE_ROWS, LAST_TILE_ROWS)],
            )

    return body(ea1, ea2, idx0, idx1)


def _tc_merge(partials, edge_attr_0, mfac):
    def body(m_ref, p0_ref, p1_ref, ea0_ref, o_ref):
        o_ref[...] = (p0_ref[...] + p1_ref[...]) * m_ref[0] + ea0_ref[...]

    return pl.pallas_call(
        body,
        grid=(1,),
        in_specs=[
            pl.BlockSpec(memory_space=pltpu.SMEM),
            pl.BlockSpec((N_NODES, D), lambda i: (0, 0)),
            pl.BlockSpec((N_NODES, D), lambda i: (1, 0)),
            pl.BlockSpec((N_NODES, D), lambda i: (0, 0)),
        ],
        out_specs=pl.BlockSpec((N_NODES, D), lambda i: (0, 0)),
        out_shape=jax.ShapeDtypeStruct((N_NODES, D), jnp.float32),
    )(mfac, partials, partials, edge_attr_0)


def kernel(edge_attr_0, edge_attr_1, edge_attr_2, edge_index_0, edge_index_1, num_nodes):
    idx0 = edge_index_0[0].reshape(NW, NCHUNK, CH)
    idx1 = edge_index_1[0].reshape(NW, NCHUNK, CH)
    partials = _sc_scatter(edge_attr_1, edge_attr_2, idx0, idx1)
    mfac = (jnp.asarray(num_nodes, jnp.int32) // N_NODES).astype(jnp.float32).reshape(1)
    return _tc_merge(partials, edge_attr_0, mfac)
